# baseline (speedup 1.0000x reference)
import functools

import jax
import jax.numpy as jnp
from jax import lax
from jax.experimental import pallas as pl
from jax.experimental.pallas import tpu as pltpu
from jax.experimental.pallas import tpu_sc as plsc

S = 7
NB = 2
NUM_CLASSES = 20
L_COORD = 5.0
L_NOOBJ = 0.5
C = 5 * NB + NUM_CLASSES
CP = C
NW = 32
LANES = 16


def _iota16():
    return lax.broadcasted_iota(jnp.int32, (LANES,), 0)


def _sqrt16(x):
    xi = lax.bitcast_convert_type(x, jnp.int32)
    yi = jnp.int32(0x5F3759DF) - lax.shift_right_arithmetic(xi, 1)
    y = lax.bitcast_convert_type(yi, jnp.float32)
    for _ in range(3):
        y = y * (jnp.float32(1.5) - jnp.float32(0.5) * x * y * y)
    s = x * y
    return jnp.float32(0.5) * (s + x / s)


@functools.lru_cache(maxsize=None)
def _sc_call(batch, nbox):
    ncell = batch * S * S
    bpw = nbox // NW
    cpw = 2 * ncell // NW
    cell_size = jnp.float32(1.0 / S)
    d = jnp.float32(14.0)

    def body(plin_hbm, noo_hbm, tflat_hbm, out_hbm,
             t_v, keys_v, idx_v, rows_v, dense_v, res_v, sem_g, sem_d):
        wid = lax.axis_index("s") * 2 + lax.axis_index("c")
        lane = _iota16()

        pltpu.sync_copy(tflat_hbm, t_v)

        def box_fields(bidx):
            base = bidx * 6
            t0 = plsc.load_gather(t_v, [base])
            t1 = plsc.load_gather(t_v, [base + 1])
            x1 = plsc.load_gather(t_v, [base + 2])
            y1 = plsc.load_gather(t_v, [base + 3])
            x2 = plsc.load_gather(t_v, [base + 4])
            y2 = plsc.load_gather(t_v, [base + 5])
            img = t0.astype(jnp.int32)
            label = t1.astype(jnp.int32)
            cx = (x1 + x2) / jnp.float32(2.0)
            cy = (y1 + y2) / jnp.float32(2.0)
            w = x2 - x1
            h = y2 - y1

            def ceil_m1(z):
                tz = z.astype(jnp.int32)
                return tz + jnp.where(z > tz.astype(jnp.float32), 1, 0) - 1

            ci = ceil_m1(cx / cell_size)
            cj = ceil_m1(cy / cell_size)
            offx = (cx - ci.astype(jnp.float32) * cell_size) / cell_size
            offy = (cy - cj.astype(jnp.float32) * cell_size) / cell_size
            rowv = img * (S * S) + ci * S + cj
            return rowv, label, offx, offy, w, h

        my_n = wid * bpw + lane
        my_cell, my_label, tx, ty, tw, th = box_fields(my_n)
        my_key = my_cell * 32 + my_label

        idx_v[...] = my_cell
        cp_rows = pltpu.async_copy(plin_hbm.at[idx_v], rows_v, sem_g)
        cp_dense = pltpu.async_copy(
            noo_hbm.at[pl.ds(wid * cpw, cpw)], dense_v, sem_d)

        for j in range(nbox // LANES):
            cellv, label, _, _, _, _ = box_fields(j * LANES + lane)
            keys_v[pl.ds(j * LANES, LANES)] = cellv * 32 + label

        UNROLL = 4

        def dedup_body(k4, carry):
            accw, acck = carry
            for u in range(UNROLL):
                k = k4 * UNROLL + u
                r = jnp.bitwise_and(k, LANES - 1)
                oidx = (k - r) + jnp.bitwise_and(lane + r, LANES - 1)
                ok = plsc.load_gather(keys_v, [oidx])
                later = oidx > my_n
                accw = accw | jnp.where(
                    later & (lax.shift_right_arithmetic(ok, 5) == my_cell), 1, 0)
                acck = acck | jnp.where(later & (ok == my_key), 1, 0)
            return accw, acck

        zero = jnp.zeros((LANES,), jnp.int32)
        accw, acck = lax.fori_loop(0, nbox // UNROLL, dedup_body, (zero, zero))
        winner = accw == 0
        ckeep = acck == 0

        cp_dense.wait()
        dacc = jnp.zeros((LANES,), jnp.float32)
        for kk in range(cpw // LANES):
            v = dense_v[pl.ds(kk * LANES, LANES)]
            dacc = dacc + v * v

        cp_rows.wait()

        def col(c):
            return plsc.load_gather(rows_v, [lane, jnp.full((LANES,), c, jnp.int32)])

        px0, py0, pw0, ph0, pc0 = col(0), col(1), col(2), col(3), col(4)
        px1, py1, pw1, ph1, pc1 = col(5), col(6), col(7), col(8), col(9)
        p_label = plsc.load_gather(rows_v, [lane, 10 + my_label])
        cls_sq = jnp.zeros((LANES,), jnp.float32)
        for c in range(10, C):
            v = col(c)
            cls_sq = cls_sq + v * v

        half = jnp.float32(0.5)
        l2x = tx / d - tw * half
        l2y = ty / d - th * half
        r2x = tx / d + tw * half
        r2y = ty / d + th * half
        area2 = (r2x - l2x) * (r2y - l2y)

        def iou_of(px, py, pw, ph):
            l1x = px / d - pw * half
            l1y = py / d - ph * half
            r1x = px / d + pw * half
            r1y = py / d + ph * half
            area1 = (r1x - l1x) * (r1y - l1y)
            wi = jnp.maximum(
                jnp.minimum(r1x, r2x) - jnp.maximum(l1x, l2x), jnp.float32(0.0))
            hi = jnp.maximum(
                jnp.minimum(r1y, r2y) - jnp.maximum(l1y, l2y), jnp.float32(0.0))
            inter = wi * hi
            return inter / (area1 + area2 - inter)

        iou0 = iou_of(px0, py0, pw0, ph0)
        iou1 = iou_of(px1, py1, pw1, ph1)
        sel = iou1 > iou0
        max_iou = jnp.where(sel, iou1, iou0)
        bx = jnp.where(sel, px1, px0)
        by = jnp.where(sel, py1, py0)
        bw = jnp.where(sel, pw1, pw0)
        bh = jnp.where(sel, ph1, ph0)
        bc = jnp.where(sel, pc1, pc0)

        dxy0 = bx - tx
        dxy1 = by - ty
        dwh0 = _sqrt16(bw) - _sqrt16(tw)
        dwh1 = _sqrt16(bh) - _sqrt16(th)
        dcon = bc - max_iou
        loc = dxy0 * dxy0 + dxy1 * dxy1 + dwh0 * dwh0 + dwh1 * dwh1
        contain = dcon * dcon
        conf_sq = pc0 * pc0 + pc1 * pc1
        per_w = (jnp.float32(L_COORD) * loc + contain + cls_sq
                 - jnp.float32(L_NOOBJ) * conf_sq)
        per_c = jnp.float32(1.0) - jnp.float32(2.0) * p_label

        acc = (jnp.where(winner, per_w, jnp.float32(0.0))
               + jnp.where(ckeep, per_c, jnp.float32(0.0))
               + jnp.float32(L_NOOBJ) * dacc)
        res_v[...] = acc * jnp.float32(1.0 / batch)
        pltpu.sync_copy(res_v, out_hbm.at[wid])

    return pl.kernel(
        body,
        out_type=jax.ShapeDtypeStruct((NW, LANES), jnp.float32),
        mesh=plsc.VectorSubcoreMesh(core_axis_name="c", subcore_axis_name="s"),
        compiler_params=pltpu.CompilerParams(
            use_tc_tiling_on_sc=False, needs_layout_passes=False),
        scratch_types=[
            pltpu.VMEM((nbox * 6,), jnp.float32),
            pltpu.VMEM((nbox,), jnp.int32),
            pltpu.VMEM((bpw,), jnp.int32),
            pltpu.VMEM((bpw, CP), jnp.float32),
            pltpu.VMEM((cpw,), jnp.float32),
            pltpu.VMEM((LANES,), jnp.float32),
            pltpu.SemaphoreType.DMA,
            pltpu.SemaphoreType.DMA,
        ],
    )


@jax.jit
def _run(pred, targets):
    batch = pred.shape[0]
    nbox = targets.shape[0]
    p2 = pred.reshape(batch * S * S, C)
    noo = p2[:, 4:10:5].reshape(-1)
    partials = _sc_call(batch, nbox)(p2, noo, targets.reshape(-1))
    return jnp.sum(partials)


def kernel(pred, targets, device=0):
    return _run(pred, targets) + jnp.asarray(device, jnp.float32) * 0.0


# speedup vs baseline: 2.1365x; 2.1365x over previous
import functools

import jax
import jax.numpy as jnp
from jax import lax
from jax.experimental import pallas as pl
from jax.experimental.pallas import tpu as pltpu
from jax.experimental.pallas import tpu_sc as plsc

S = 7
NB = 2
NUM_CLASSES = 20
L_COORD = 5.0
L_NOOBJ = 0.5
C = 5 * NB + NUM_CLASSES
CP = 128
SP = 8
RPI = S * SP
NW = 32
LANES = 16


def _iota16():
    return lax.broadcasted_iota(jnp.int32, (LANES,), 0)


def _sqrt16(x):
    xi = lax.bitcast_convert_type(x, jnp.int32)
    yi = jnp.int32(0x5F3759DF) - lax.shift_right_arithmetic(xi, 1)
    y = lax.bitcast_convert_type(yi, jnp.float32)
    for _ in range(3):
        y = y * (jnp.float32(1.5) - jnp.float32(0.5) * x * y * y)
    s = x * y
    return jnp.float32(0.5) * (s + x / s)


@functools.lru_cache(maxsize=None)
def _sc_call(batch, nbox):
    nrow = batch * RPI
    bpw = nbox // NW
    rpw = nrow // NW
    cell_size = jnp.float32(1.0 / S)
    d = jnp.float32(14.0)

    def body(plin_hbm, tflat_hbm, out_hbm,
             t_v, keys_v, idx_v, rows_v, dense_v, res_v, sem_g, sem_d):
        wid = lax.axis_index("s") * 2 + lax.axis_index("c")
        lane = _iota16()

        pltpu.sync_copy(tflat_hbm, t_v)

        def box_fields(bidx):
            base = bidx * 6
            t0 = plsc.load_gather(t_v, [base])
            t1 = plsc.load_gather(t_v, [base + 1])
            x1 = plsc.load_gather(t_v, [base + 2])
            y1 = plsc.load_gather(t_v, [base + 3])
            x2 = plsc.load_gather(t_v, [base + 4])
            y2 = plsc.load_gather(t_v, [base + 5])
            img = t0.astype(jnp.int32)
            label = t1.astype(jnp.int32)
            cx = (x1 + x2) / jnp.float32(2.0)
            cy = (y1 + y2) / jnp.float32(2.0)
            w = x2 - x1
            h = y2 - y1

            def ceil_m1(z):
                tz = z.astype(jnp.int32)
                return tz + jnp.where(z > tz.astype(jnp.float32), 1, 0) - 1

            ci = ceil_m1(cx / cell_size)
            cj = ceil_m1(cy / cell_size)
            offx = (cx - ci.astype(jnp.float32) * cell_size) / cell_size
            offy = (cy - cj.astype(jnp.float32) * cell_size) / cell_size
            rowv = img * RPI + ci * SP + cj
            return rowv, label, offx, offy, w, h

        my_n = wid * bpw + lane
        my_cell, my_label, tx, ty, tw, th = box_fields(my_n)
        my_key = my_cell * 32 + my_label

        idx_v[...] = my_cell
        cp_rows = pltpu.async_copy(plin_hbm.at[idx_v], rows_v, sem_g)
        cp_dense = pltpu.async_copy(
            plin_hbm.at[pl.ds(wid * rpw, rpw)], dense_v, sem_d)

        for j in range(nbox // LANES):
            cellv, label, _, _, _, _ = box_fields(j * LANES + lane)
            keys_v[pl.ds(j * LANES, LANES)] = cellv * 32 + label

        UNROLL = 4

        def dedup_body(k4, carry):
            accw, acck = carry
            for u in range(UNROLL):
                k = k4 * UNROLL + u
                r = jnp.bitwise_and(k, LANES - 1)
                oidx = (k - r) + jnp.bitwise_and(lane + r, LANES - 1)
                ok = plsc.load_gather(keys_v, [oidx])
                later = oidx > my_n
                accw = accw | jnp.where(
                    later & (lax.shift_right_arithmetic(ok, 5) == my_cell), 1, 0)
                acck = acck | jnp.where(later & (ok == my_key), 1, 0)
            return accw, acck

        zero = jnp.zeros((LANES,), jnp.int32)
        accw, acck = lax.fori_loop(0, nbox // UNROLL, dedup_body, (zero, zero))
        winner = accw == 0
        ckeep = acck == 0

        cp_dense.wait()
        col4 = jnp.full((LANES,), 4, jnp.int32)
        col9 = jnp.full((LANES,), 9, jnp.int32)
        dacc = jnp.zeros((LANES,), jnp.float32)
        for kk in range(rpw // LANES):
            row = kk * LANES + lane
            g4 = plsc.load_gather(dense_v, [row, col4])
            g9 = plsc.load_gather(dense_v, [row, col9])
            dacc = dacc + g4 * g4 + g9 * g9

        cp_rows.wait()

        def col(c):
            return plsc.load_gather(rows_v, [lane, jnp.full((LANES,), c, jnp.int32)])

        px0, py0, pw0, ph0, pc0 = col(0), col(1), col(2), col(3), col(4)
        px1, py1, pw1, ph1, pc1 = col(5), col(6), col(7), col(8), col(9)
        p_label = plsc.load_gather(rows_v, [lane, 10 + my_label])
        cls_sq = jnp.zeros((LANES,), jnp.float32)
        for c in range(10, C):
            v = col(c)
            cls_sq = cls_sq + v * v

        half = jnp.float32(0.5)
        l2x = tx / d - tw * half
        l2y = ty / d - th * half
        r2x = tx / d + tw * half
        r2y = ty / d + th * half
        area2 = (r2x - l2x) * (r2y - l2y)

        def iou_of(px, py, pw, ph):
            l1x = px / d - pw * half
            l1y = py / d - ph * half
            r1x = px / d + pw * half
            r1y = py / d + ph * half
            area1 = (r1x - l1x) * (r1y - l1y)
            wi = jnp.maximum(
                jnp.minimum(r1x, r2x) - jnp.maximum(l1x, l2x), jnp.float32(0.0))
            hi = jnp.maximum(
                jnp.minimum(r1y, r2y) - jnp.maximum(l1y, l2y), jnp.float32(0.0))
            inter = wi * hi
            return inter / (area1 + area2 - inter)

        iou0 = iou_of(px0, py0, pw0, ph0)
        iou1 = iou_of(px1, py1, pw1, ph1)
        sel = iou1 > iou0
        max_iou = jnp.where(sel, iou1, iou0)
        bx = jnp.where(sel, px1, px0)
        by = jnp.where(sel, py1, py0)
        bw = jnp.where(sel, pw1, pw0)
        bh = jnp.where(sel, ph1, ph0)
        bc = jnp.where(sel, pc1, pc0)

        dxy0 = bx - tx
        dxy1 = by - ty
        dwh0 = _sqrt16(bw) - _sqrt16(tw)
        dwh1 = _sqrt16(bh) - _sqrt16(th)
        dcon = bc - max_iou
        loc = dxy0 * dxy0 + dxy1 * dxy1 + dwh0 * dwh0 + dwh1 * dwh1
        contain = dcon * dcon
        conf_sq = pc0 * pc0 + pc1 * pc1
        per_w = (jnp.float32(L_COORD) * loc + contain + cls_sq
                 - jnp.float32(L_NOOBJ) * conf_sq)
        per_c = jnp.float32(1.0) - jnp.float32(2.0) * p_label

        acc = (jnp.where(winner, per_w, jnp.float32(0.0))
               + jnp.where(ckeep, per_c, jnp.float32(0.0))
               + jnp.float32(L_NOOBJ) * dacc)
        res_v[...] = acc * jnp.float32(1.0 / batch)
        pltpu.sync_copy(res_v, out_hbm.at[wid])

    return pl.kernel(
        body,
        out_type=jax.ShapeDtypeStruct((NW, LANES), jnp.float32),
        mesh=plsc.VectorSubcoreMesh(core_axis_name="c", subcore_axis_name="s"),
        compiler_params=pltpu.CompilerParams(
            use_tc_tiling_on_sc=False, needs_layout_passes=False),
        scratch_types=[
            pltpu.VMEM((nbox * 6,), jnp.float32),
            pltpu.VMEM((nbox,), jnp.int32),
            pltpu.VMEM((bpw,), jnp.int32),
            pltpu.VMEM((bpw, CP), jnp.float32),
            pltpu.VMEM((rpw, CP), jnp.float32),
            pltpu.VMEM((LANES,), jnp.float32),
            pltpu.SemaphoreType.DMA,
            pltpu.SemaphoreType.DMA,
        ],
    )


@jax.jit
def _run(pred, targets):
    batch = pred.shape[0]
    nbox = targets.shape[0]
    xp = jnp.zeros((batch, S, SP, CP), pred.dtype).at[:, :, :S, :C].set(pred)
    plin = xp.reshape(batch * RPI, CP)
    partials = _sc_call(batch, nbox)(plin, targets.reshape(-1))
    return jnp.sum(partials)


def kernel(pred, targets, device=0):
    return _run(pred, targets) + jnp.asarray(device, jnp.float32) * 0.0


# speedup vs baseline: 2.1848x; 1.0226x over previous
import functools

import jax
import jax.numpy as jnp
from jax import lax
from jax.experimental import pallas as pl
from jax.experimental.pallas import tpu as pltpu
from jax.experimental.pallas import tpu_sc as plsc

S = 7
NB = 2
NUM_CLASSES = 20
L_COORD = 5.0
L_NOOBJ = 0.5
C = 5 * NB + NUM_CLASSES
CP = 128
SP = 8
RPI = S * SP
NW = 32
LANES = 16


def _iota16():
    return lax.broadcasted_iota(jnp.int32, (LANES,), 0)


def _sqrt16(x):
    xi = lax.bitcast_convert_type(x, jnp.int32)
    yi = jnp.int32(0x5F3759DF) - lax.shift_right_arithmetic(xi, 1)
    y = lax.bitcast_convert_type(yi, jnp.float32)
    for _ in range(3):
        y = y * (jnp.float32(1.5) - jnp.float32(0.5) * x * y * y)
    s = x * y
    return jnp.float32(0.5) * (s + x / s)


@functools.lru_cache(maxsize=None)
def _sc_call(batch, nbox):
    nrow = batch * RPI
    bpw = nbox // NW
    rpw = nrow // NW
    cell_size = jnp.float32(1.0 / S)
    d = jnp.float32(14.0)

    def body(plin_hbm, tflat_hbm, out_hbm,
             t_v, keys_v, idx_v, rows_v, dense_v, res_v, sem_g, sem_d):
        wid = lax.axis_index("s") * 2 + lax.axis_index("c")
        lane = _iota16()

        pltpu.sync_copy(tflat_hbm, t_v)

        def box_fields(bidx):
            base = bidx * 6
            t0 = plsc.load_gather(t_v, [base])
            t1 = plsc.load_gather(t_v, [base + 1])
            x1 = plsc.load_gather(t_v, [base + 2])
            y1 = plsc.load_gather(t_v, [base + 3])
            x2 = plsc.load_gather(t_v, [base + 4])
            y2 = plsc.load_gather(t_v, [base + 5])
            img = t0.astype(jnp.int32)
            label = t1.astype(jnp.int32)
            cx = (x1 + x2) / jnp.float32(2.0)
            cy = (y1 + y2) / jnp.float32(2.0)
            w = x2 - x1
            h = y2 - y1

            def ceil_m1(z):
                tz = z.astype(jnp.int32)
                return tz + jnp.where(z > tz.astype(jnp.float32), 1, 0) - 1

            ci = ceil_m1(cx / cell_size)
            cj = ceil_m1(cy / cell_size)
            offx = (cx - ci.astype(jnp.float32) * cell_size) / cell_size
            offy = (cy - cj.astype(jnp.float32) * cell_size) / cell_size
            rowv = img * RPI + ci * SP + cj
            return rowv, label, offx, offy, w, h

        my_n = wid * bpw + lane
        my_cell, my_label, tx, ty, tw, th = box_fields(my_n)
        my_key = my_cell * 32 + my_label

        idx_v[...] = my_cell
        cp_rows = pltpu.async_copy(plin_hbm.at[idx_v], rows_v, sem_g)
        cp_dense = pltpu.async_copy(
            plin_hbm.at[pl.ds(wid * rpw, rpw), pl.ds(0, 32)], dense_v, sem_d)

        for j in range(nbox // LANES):
            cellv, label, _, _, _, _ = box_fields(j * LANES + lane)
            keys_v[pl.ds(j * LANES, LANES)] = cellv * 32 + label

        UNROLL = 4

        def dedup_body(k4, carry):
            accw, acck = carry
            for u in range(UNROLL):
                k = k4 * UNROLL + u
                r = jnp.bitwise_and(k, LANES - 1)
                oidx = (k - r) + jnp.bitwise_and(lane + r, LANES - 1)
                ok = plsc.load_gather(keys_v, [oidx])
                later = oidx > my_n
                accw = accw | jnp.where(
                    later & (lax.shift_right_arithmetic(ok, 5) == my_cell), 1, 0)
                acck = acck | jnp.where(later & (ok == my_key), 1, 0)
            return accw, acck

        zero = jnp.zeros((LANES,), jnp.int32)
        accw, acck = lax.fori_loop(0, nbox // UNROLL, dedup_body, (zero, zero))
        winner = accw == 0
        ckeep = acck == 0

        cp_dense.wait()
        col4 = jnp.full((LANES,), 4, jnp.int32)
        col9 = jnp.full((LANES,), 9, jnp.int32)
        dacc = jnp.zeros((LANES,), jnp.float32)
        for kk in range(rpw // LANES):
            row = kk * LANES + lane
            g4 = plsc.load_gather(dense_v, [row, col4])
            g9 = plsc.load_gather(dense_v, [row, col9])
            dacc = dacc + g4 * g4 + g9 * g9

        cp_rows.wait()

        def col(c):
            return plsc.load_gather(rows_v, [lane, jnp.full((LANES,), c, jnp.int32)])

        px0, py0, pw0, ph0, pc0 = col(0), col(1), col(2), col(3), col(4)
        px1, py1, pw1, ph1, pc1 = col(5), col(6), col(7), col(8), col(9)
        p_label = plsc.load_gather(rows_v, [lane, 10 + my_label])
        cls_sq = jnp.zeros((LANES,), jnp.float32)
        for c in range(10, C):
            v = col(c)
            cls_sq = cls_sq + v * v

        half = jnp.float32(0.5)
        l2x = tx / d - tw * half
        l2y = ty / d - th * half
        r2x = tx / d + tw * half
        r2y = ty / d + th * half
        area2 = (r2x - l2x) * (r2y - l2y)

        def iou_of(px, py, pw, ph):
            l1x = px / d - pw * half
            l1y = py / d - ph * half
            r1x = px / d + pw * half
            r1y = py / d + ph * half
            area1 = (r1x - l1x) * (r1y - l1y)
            wi = jnp.maximum(
                jnp.minimum(r1x, r2x) - jnp.maximum(l1x, l2x), jnp.float32(0.0))
            hi = jnp.maximum(
                jnp.minimum(r1y, r2y) - jnp.maximum(l1y, l2y), jnp.float32(0.0))
            inter = wi * hi
            return inter / (area1 + area2 - inter)

        iou0 = iou_of(px0, py0, pw0, ph0)
        iou1 = iou_of(px1, py1, pw1, ph1)
        sel = iou1 > iou0
        max_iou = jnp.where(sel, iou1, iou0)
        bx = jnp.where(sel, px1, px0)
        by = jnp.where(sel, py1, py0)
        bw = jnp.where(sel, pw1, pw0)
        bh = jnp.where(sel, ph1, ph0)
        bc = jnp.where(sel, pc1, pc0)

        dxy0 = bx - tx
        dxy1 = by - ty
        dwh0 = _sqrt16(bw) - _sqrt16(tw)
        dwh1 = _sqrt16(bh) - _sqrt16(th)
        dcon = bc - max_iou
        loc = dxy0 * dxy0 + dxy1 * dxy1 + dwh0 * dwh0 + dwh1 * dwh1
        contain = dcon * dcon
        conf_sq = pc0 * pc0 + pc1 * pc1
        per_w = (jnp.float32(L_COORD) * loc + contain + cls_sq
                 - jnp.float32(L_NOOBJ) * conf_sq)
        per_c = jnp.float32(1.0) - jnp.float32(2.0) * p_label

        acc = (jnp.where(winner, per_w, jnp.float32(0.0))
               + jnp.where(ckeep, per_c, jnp.float32(0.0))
               + jnp.float32(L_NOOBJ) * dacc)
        res_v[...] = acc * jnp.float32(1.0 / batch)
        pltpu.sync_copy(res_v, out_hbm.at[wid])

    return pl.kernel(
        body,
        out_type=jax.ShapeDtypeStruct((NW, LANES), jnp.float32),
        mesh=plsc.VectorSubcoreMesh(core_axis_name="c", subcore_axis_name="s"),
        compiler_params=pltpu.CompilerParams(
            use_tc_tiling_on_sc=False, needs_layout_passes=False),
        scratch_types=[
            pltpu.VMEM((nbox * 6,), jnp.float32),
            pltpu.VMEM((nbox,), jnp.int32),
            pltpu.VMEM((bpw,), jnp.int32),
            pltpu.VMEM((bpw, CP), jnp.float32),
            pltpu.VMEM((rpw, 32), jnp.float32),
            pltpu.VMEM((LANES,), jnp.float32),
            pltpu.SemaphoreType.DMA,
            pltpu.SemaphoreType.DMA,
        ],
    )


@jax.jit
def _run(pred, targets):
    batch = pred.shape[0]
    nbox = targets.shape[0]
    xp = jnp.pad(pred, ((0, 0), (0, 0), (0, SP - S), (0, CP - C)))
    plin = xp.reshape(batch * RPI, CP)
    partials = _sc_call(batch, nbox)(plin, targets.reshape(-1))
    return jnp.sum(partials)


def kernel(pred, targets, device=0):
    return _run(pred, targets) + jnp.asarray(device, jnp.float32) * 0.0
